# baseline (speedup 1.0000x reference)
import functools

import jax
import jax.numpy as jnp
from jax.experimental import pallas as pl


def _matmul_body(x_ref, w_ref, b_ref, o_ref):
    o_ref[...] = (
        jnp.dot(x_ref[...], w_ref[...], preferred_element_type=jnp.float32)
        + b_ref[...]
    )


@functools.partial(jax.jit, static_argnames=("block_rows",))
def _project(x, W, b, block_rows=1000):
    N, din = x.shape
    dout = W.shape[1]
    grid = (N // block_rows,)
    return pl.pallas_call(
        _matmul_body,
        grid=grid,
        in_specs=[
            pl.BlockSpec((block_rows, din), lambda i: (i, 0)),
            pl.BlockSpec((din, dout), lambda i: (0, 0)),
            pl.BlockSpec((1, dout), lambda i: (0, 0)),
        ],
        out_specs=pl.BlockSpec((block_rows, dout), lambda i: (i, 0)),
        out_shape=jax.ShapeDtypeStruct((N, dout), jnp.float32),
    )(x, W, b.reshape(1, dout))


def _gatv2(x, src, dst, Wl, bl, Wr, br, att, bias):
    N = x.shape[0]
    h, dc = att.shape
    W2 = jnp.concatenate([Wl, Wr], axis=1)
    b2 = jnp.concatenate([bl, br], axis=0)
    proj = _project(x, W2, b2)
    xl = proj[:, : h * dc].reshape(N, h, dc)
    xr = proj[:, h * dc :].reshape(N, h, dc)
    e = jax.nn.leaky_relu(xl[src] + xr[dst], negative_slope=0.2)
    logits = jnp.einsum("ehd,hd->eh", e, att)
    m = jax.ops.segment_max(logits, dst, num_segments=N)
    m = jnp.where(jnp.isfinite(m), m, 0.0)
    ex = jnp.exp(logits - m[dst])
    den = jax.ops.segment_sum(ex, dst, num_segments=N)
    alpha = ex / (den[dst] + 1e-16)
    out = jax.ops.segment_sum(xl[src] * alpha[:, :, None], dst, num_segments=N)
    return out.reshape(N, h * dc) + bias


def kernel(x, adj, params):
    adj = adj.astype(jnp.int32)
    src, dst = adj[0], adj[1]
    h = x
    for i in range(5):
        h = _gatv2(h, src, dst, *params[i])
        if i < 4:
            h = jax.nn.relu(h)
    h = jax.nn.softmax(h, axis=1)
    h1 = h
    for i in range(5, 10):
        h1 = _gatv2(h1, src, dst, *params[i])
        if i < 9:
            h1 = jax.nn.relu(h1)
    h1 = jax.nn.softmax(h1, axis=1)
    return (h1, h)


# speedup vs baseline: 6.0756x; 6.0756x over previous
import functools

import jax
import jax.numpy as jnp
from jax import lax
from jax.experimental import pallas as pl
from jax.experimental.pallas import tpu as pltpu
from jax.experimental.pallas import tpu_sc as plsc

N_NODES = 10000
N_EDGES = 160000
NC, NS, LANES = 2, 16, 16
NW = NC * NS
NPW = 313
N_PAD = NW * NPW
EDGE_B = 128
E_PAD = N_EDGES + 2 * EDGE_B
EX_SPACING = EDGE_B + 16
EX_ROWS = N_EDGES + EX_SPACING * NW + 2 * EDGE_B


def _proj_body_plain(x_ref, w_ref, b_ref, o_ref):
    o_ref[...] = (
        jnp.dot(x_ref[...], w_ref[...], preferred_element_type=jnp.float32)
        + b_ref[...]
    )


def _proj_body_relu(x_ref, w_ref, b_ref, pb_ref, o_ref):
    a = jnp.maximum(x_ref[...] + pb_ref[...], 0.0)
    o_ref[...] = (
        jnp.dot(a, w_ref[...], preferred_element_type=jnp.float32) + b_ref[...]
    )


def _project(x, W, b, prev_bias=None, block_rows=1000):
    N, din = x.shape
    dout = W.shape[1]
    grid = (N // block_rows,)
    in_specs = [
        pl.BlockSpec((block_rows, din), lambda i: (i, 0)),
        pl.BlockSpec((din, dout), lambda i: (0, 0)),
        pl.BlockSpec((1, dout), lambda i: (0, 0)),
    ]
    args = [x, W, b.reshape(1, dout)]
    body = _proj_body_plain
    if prev_bias is not None:
        in_specs.append(pl.BlockSpec((1, din), lambda i: (0, 0)))
        args.append(prev_bias.reshape(1, din))
        body = _proj_body_relu
    return pl.pallas_call(
        body,
        grid=grid,
        in_specs=in_specs,
        out_specs=pl.BlockSpec((block_rows, dout), lambda i: (i, 0)),
        out_shape=jax.ShapeDtypeStruct((N, dout), jnp.float32),
    )(*args)


def _softmax_body(x_ref, b_ref, o_ref):
    z = x_ref[...] + b_ref[...]
    z = z - jnp.max(z, axis=1, keepdims=True)
    e = jnp.exp(z)
    o_ref[...] = e / jnp.sum(e, axis=1, keepdims=True)


def _softmax_bias(x, b, block_rows=1000):
    N, d = x.shape
    return pl.pallas_call(
        _softmax_body,
        grid=(N // block_rows,),
        in_specs=[
            pl.BlockSpec((block_rows, d), lambda i: (i, 0)),
            pl.BlockSpec((1, d), lambda i: (0, 0)),
        ],
        out_specs=pl.BlockSpec((block_rows, d), lambda i: (i, 0)),
        out_shape=jax.ShapeDtypeStruct((N, d), jnp.float32),
    )(x, b.reshape(1, d))


@functools.cache
def _edge_kernel(h, dc, nc, h_c):
    C = h_c * dc
    n_groups = C // LANES
    gph = dc // LANES

    mesh = plsc.VectorSubcoreMesh(
        core_axis_name="c", subcore_axis_name="s",
        num_cores=NC, num_subcores=NS)

    def body(xl_hbm, xr_hbm, att_hbm, src_hbm, dst_hbm, ebnd_hbm,
             out_hbm, ex_hbm,
             xr_l, out_l, xl_b, m_l, den_l, lg, al, att_v, sidx, didx,
             ebv, sem):
        wid = lax.axis_index("s") * NC + lax.axis_index("c")
        n0 = wid * NPW
        fbase = pl.multiple_of(n0 * C, 8)
        pltpu.sync_copy(ebnd_hbm, ebv.at[pl.ds(0, 40)])
        e0 = ebv[pl.ds(wid, LANES)][0]
        e1 = ebv[pl.ds(wid + 1, LANES)][0]
        a0 = (e0 // 8) * 8
        nb = (e1 - a0 + EDGE_B - 1) // EDGE_B
        exbase = ((e0 + 7) // 8) * 8 + EX_SPACING * wid
        iota = lax.broadcasted_iota(jnp.int32, (LANES,), 0)
        lanemask = iota < h_c

        for ci in range(nc):
            pltpu.sync_copy(
                xr_hbm.at[ci, pl.ds(fbase, NPW * C)], xr_l)
            pltpu.sync_copy(att_hbm.at[ci], att_v)

            def _zero_out(j, _):
                out_l[pl.ds(j * LANES, LANES)] = jnp.zeros(
                    (LANES,), jnp.float32)
                return 0
            lax.fori_loop(0, NPW * C // LANES, _zero_out, 0)

            def _init_md(j, _):
                m_l[pl.ds(j * LANES, LANES)] = jnp.full(
                    (LANES,), -3.0e38, jnp.float32)
                den_l[pl.ds(j * LANES, LANES)] = jnp.zeros(
                    (LANES,), jnp.float32)
                return 0
            lax.fori_loop(0, NPW, _init_md, 0)

            def _batch1(b, _):
                base = pl.multiple_of(a0 + b * EDGE_B, 8)
                pltpu.sync_copy(src_hbm.at[pl.ds(base, EDGE_B)], sidx)
                pltpu.sync_copy(dst_hbm.at[pl.ds(base, EDGE_B)],
                                didx.at[pl.ds(0, EDGE_B)])
                pltpu.async_copy(xl_hbm.at[ci].at[sidx], xl_b, sem).wait()
                lo = jnp.maximum(e0 - base, 0)
                hi = jnp.minimum(e1 - base, EDGE_B)

                def _edge1(i, _):
                    dloc = didx[pl.ds(i, LANES)][0] - n0
                    parts = []
                    for g in range(n_groups):
                        v = (xl_b[i, pl.ds(g * LANES, LANES)]
                             + xr_l[pl.ds(dloc * C + g * LANES, LANES)])
                        v = jnp.maximum(v, 0.0) + 0.2 * jnp.minimum(v, 0.0)
                        parts.append(v * att_v[pl.ds(g * LANES, LANES)])
                    lvec = jnp.zeros((LANES,), jnp.float32)
                    for hh in range(h_c):
                        acc = parts[hh * gph]
                        for g in range(1, gph):
                            acc = acc + parts[hh * gph + g]
                        lvec = jnp.where(iota == hh, jnp.sum(acc), lvec)
                    lg[i, :] = lvec
                    idx = dloc * LANES + iota
                    mvec = plsc.load_gather(m_l, [idx])
                    plsc.store_scatter(
                        m_l, [idx], jnp.maximum(mvec, lvec), mask=lanemask)
                    return 0
                lax.fori_loop(lo, hi, _edge1, 0)
                exrow = pl.multiple_of(exbase + b * EDGE_B, 8)
                pltpu.sync_copy(lg, ex_hbm.at[pl.ds(exrow, EDGE_B), :])
                return 0
            lax.fori_loop(0, nb, _batch1, 0)

            def _batch2(b, _):
                base = pl.multiple_of(a0 + b * EDGE_B, 8)
                pltpu.sync_copy(dst_hbm.at[pl.ds(base, EDGE_B)],
                                didx.at[pl.ds(0, EDGE_B)])
                exrow = pl.multiple_of(exbase + b * EDGE_B, 8)
                pltpu.sync_copy(ex_hbm.at[pl.ds(exrow, EDGE_B), :], lg)
                lo = jnp.maximum(e0 - base, 0)
                hi = jnp.minimum(e1 - base, EDGE_B)

                def _edge2(i, _):
                    dloc = didx[pl.ds(i, LANES)][0] - n0
                    idx = dloc * LANES + iota
                    mvec = plsc.load_gather(m_l, [idx])
                    ex = jnp.exp(lg[i, :] - mvec)
                    lg[i, :] = ex
                    plsc.addupdate_scatter(den_l, [idx], ex, mask=lanemask)
                    return 0
                lax.fori_loop(lo, hi, _edge2, 0)
                pltpu.sync_copy(lg, ex_hbm.at[pl.ds(exrow, EDGE_B), :])
                return 0
            lax.fori_loop(0, nb, _batch2, 0)

            def _batch3(b, _):
                base = pl.multiple_of(a0 + b * EDGE_B, 8)
                pltpu.sync_copy(src_hbm.at[pl.ds(base, EDGE_B)], sidx)
                pltpu.sync_copy(dst_hbm.at[pl.ds(base, EDGE_B)],
                                didx.at[pl.ds(0, EDGE_B)])
                pltpu.async_copy(xl_hbm.at[ci].at[sidx], xl_b, sem).wait()
                exrow = pl.multiple_of(exbase + b * EDGE_B, 8)
                pltpu.sync_copy(ex_hbm.at[pl.ds(exrow, EDGE_B), :], lg)
                lo = jnp.maximum(e0 - base, 0)
                hi = jnp.minimum(e1 - base, EDGE_B)

                def _edge3(i, _):
                    dloc = didx[pl.ds(i, LANES)][0] - n0
                    idx = dloc * LANES + iota
                    den = plsc.load_gather(den_l, [idx])
                    al[:] = lg[i, :] / (den + 1e-16)
                    for g in range(n_groups):
                        head = (g * LANES) // dc
                        av = plsc.load_gather(
                            al, [jnp.full((LANES,), head, jnp.int32)])
                        contrib = xl_b[i, pl.ds(g * LANES, LANES)] * av
                        plsc.addupdate(
                            out_l.at[pl.ds(dloc * C + g * LANES, LANES)],
                            contrib)
                    return 0
                lax.fori_loop(lo, hi, _edge3, 0)
                return 0
            lax.fori_loop(0, nb, _batch3, 0)

            pltpu.sync_copy(out_l, out_hbm.at[ci, pl.ds(fbase, NPW * C)])

    return pl.kernel(
        body,
        out_type=(
            jax.ShapeDtypeStruct((nc, N_PAD * C), jnp.float32),
            jax.ShapeDtypeStruct((EX_ROWS, LANES), jnp.float32),
        ),
        mesh=mesh,
        compiler_params=pltpu.CompilerParams(
            needs_layout_passes=False, use_tc_tiling_on_sc=False),
        scratch_types=(
            pltpu.VMEM((NPW * C,), jnp.float32),
            pltpu.VMEM((NPW * C,), jnp.float32),
            pltpu.VMEM((EDGE_B, C), jnp.float32),
            pltpu.VMEM((NPW * LANES,), jnp.float32),
            pltpu.VMEM((NPW * LANES,), jnp.float32),
            pltpu.VMEM((EDGE_B, LANES), jnp.float32),
            pltpu.VMEM((LANES,), jnp.float32),
            pltpu.VMEM((128,), jnp.float32),
            pltpu.VMEM((EDGE_B,), jnp.int32),
            pltpu.VMEM((EDGE_B + LANES,), jnp.int32),
            pltpu.VMEM((56,), jnp.int32),
            pltpu.SemaphoreType.DMA,
        ),
    )


_CHUNKS = {
    (16, 32): (4, 4),
    (25, 16): (5, 5),
    (16, 16): (2, 8),
    (1, 32): (1, 1),
    (1, 128): (1, 1),
}


def _gat_layer(act, src_s, dst_s, ebounds, Wl, bl, Wr, br, att, bias):
    h, dc = att.shape
    hdc = h * dc
    nc, h_c = _CHUNKS[(h, dc)]
    C = h_c * dc

    W2 = jnp.concatenate([Wl, Wr], axis=1)
    b2 = jnp.concatenate([bl, br], axis=0)
    proj = _project(act, W2, b2, prev_bias=bias if bias is not None else None)

    proj = jnp.pad(proj, ((0, N_PAD - N_NODES), (0, 0)))
    xl = proj[:, :hdc].reshape(N_PAD, nc, C).transpose(1, 0, 2)
    xr = proj[:, hdc:].reshape(N_PAD, nc, C).transpose(1, 0, 2)
    att_e = att.reshape(nc, C)
    att_e = jnp.pad(att_e, ((0, 0), (0, 128 - C)))

    kern = _edge_kernel(h, dc, nc, h_c)
    out_ch, _ = kern(
        xl, xr.reshape(nc, N_PAD * C), att_e,
        src_s, dst_s, ebounds)
    out = out_ch.reshape(nc, N_PAD, C).transpose(1, 0, 2).reshape(N_PAD, hdc)
    return out[:N_NODES]


def kernel(x, adj, params):
    adj = adj.astype(jnp.int32)
    src, dst = adj[0], adj[1]

    combo = jnp.sort(dst * 16384 + src)
    dst_s = (combo >> 14).astype(jnp.int32)
    src_s = (combo & 16383).astype(jnp.int32)
    ebounds = jnp.searchsorted(
        dst_s, jnp.arange(NW + 1, dtype=jnp.int32) * NPW).astype(jnp.int32)
    ebounds = jnp.pad(ebounds, (0, 40 - (NW + 1)))
    src_s = jnp.pad(src_s, (0, E_PAD - N_EDGES))
    dst_s = jnp.pad(dst_s, (0, E_PAD - N_EDGES),
                    constant_values=N_NODES - 1)

    act = x
    prev_bias = None
    h_out = None
    for i in range(10):
        Wl, bl, Wr, br, att, bias = params[i]
        if i == 0:
            agg = _gat_layer(act, src_s, dst_s, ebounds,
                             Wl, bl, Wr, br, att, None)
        elif i == 5:
            agg = _gat_layer(h_out, src_s, dst_s, ebounds,
                             Wl, bl, Wr, br, att, None)
        else:
            agg = _gat_layer(agg, src_s, dst_s, ebounds,
                             Wl, bl, Wr, br, att, prev_bias)
        prev_bias = bias
        if i == 4:
            h_out = _softmax_bias(agg, bias)
        if i == 9:
            h1_out = _softmax_bias(agg, bias)
    return (h1_out, h_out)


# speedup vs baseline: 7.0512x; 1.1606x over previous
import functools

import jax
import jax.numpy as jnp
from jax import lax
from jax.experimental import pallas as pl
from jax.experimental.pallas import tpu as pltpu
from jax.experimental.pallas import tpu_sc as plsc

N_NODES = 10000
N_EDGES = 160000
NC, NS, LANES = 2, 16, 16
NW = NC * NS
NPW = 313
N_PAD = NW * NPW
EDGE_B = 128
E_PAD = N_EDGES + 2 * EDGE_B
EX_SPACING = EDGE_B + 16
EX_ROWS = N_EDGES + EX_SPACING * NW + 2 * EDGE_B


def _proj_body_plain(x_ref, w_ref, b_ref, o_ref):
    o_ref[...] = (
        jnp.dot(x_ref[...], w_ref[...], preferred_element_type=jnp.float32)
        + b_ref[...]
    )


def _proj_body_relu(x_ref, w_ref, b_ref, pb_ref, o_ref):
    a = jnp.maximum(x_ref[...] + pb_ref[...], 0.0)
    o_ref[...] = (
        jnp.dot(a, w_ref[...], preferred_element_type=jnp.float32) + b_ref[...]
    )


def _project(x, W, b, prev_bias=None, block_rows=1000):
    N, din = x.shape
    dout = W.shape[1]
    grid = (N // block_rows,)
    in_specs = [
        pl.BlockSpec((block_rows, din), lambda i: (i, 0)),
        pl.BlockSpec((din, dout), lambda i: (0, 0)),
        pl.BlockSpec((1, dout), lambda i: (0, 0)),
    ]
    args = [x, W, b.reshape(1, dout)]
    body = _proj_body_plain
    if prev_bias is not None:
        in_specs.append(pl.BlockSpec((1, din), lambda i: (0, 0)))
        args.append(prev_bias.reshape(1, din))
        body = _proj_body_relu
    return pl.pallas_call(
        body,
        grid=grid,
        in_specs=in_specs,
        out_specs=pl.BlockSpec((block_rows, dout), lambda i: (i, 0)),
        out_shape=jax.ShapeDtypeStruct((N, dout), jnp.float32),
    )(*args)


def _softmax_body(x_ref, b_ref, o_ref):
    z = x_ref[...] + b_ref[...]
    z = z - jnp.max(z, axis=1, keepdims=True)
    e = jnp.exp(z)
    o_ref[...] = e / jnp.sum(e, axis=1, keepdims=True)


def _softmax_bias(x, b, block_rows=1000):
    N, d = x.shape
    return pl.pallas_call(
        _softmax_body,
        grid=(N // block_rows,),
        in_specs=[
            pl.BlockSpec((block_rows, d), lambda i: (i, 0)),
            pl.BlockSpec((1, d), lambda i: (0, 0)),
        ],
        out_specs=pl.BlockSpec((block_rows, d), lambda i: (i, 0)),
        out_shape=jax.ShapeDtypeStruct((N, d), jnp.float32),
    )(x, b.reshape(1, d))


@functools.cache
def _edge_kernel(h, dc, nc, h_c):
    C = h_c * dc
    n_groups = C // LANES
    gph = dc // LANES

    mesh = plsc.VectorSubcoreMesh(
        core_axis_name="c", subcore_axis_name="s",
        num_cores=NC, num_subcores=NS)

    def body(xl_hbm, xr_hbm, att_hbm, src_hbm, dst_hbm, ebnd_hbm,
             out_hbm, ex_hbm,
             xr_l, out_l, xl_b, m_l, den_l, lg, al, att_v, sidx, didx,
             ebv, sem):
        wid = lax.axis_index("s") * NC + lax.axis_index("c")
        n0 = wid * NPW
        fbase = pl.multiple_of(n0 * C, 8)
        pltpu.sync_copy(ebnd_hbm, ebv.at[pl.ds(0, 40)])
        e0 = ebv[pl.ds(wid, LANES)][0]
        e1 = ebv[pl.ds(wid + 1, LANES)][0]
        a0 = (e0 // 8) * 8
        nb = (e1 - a0 + EDGE_B - 1) // EDGE_B
        exbase = ((e0 + 7) // 8) * 8 + EX_SPACING * wid
        iota = lax.broadcasted_iota(jnp.int32, (LANES,), 0)
        lanemask = iota < h_c

        for ci in range(nc):
            pltpu.sync_copy(
                xr_hbm.at[ci, pl.ds(fbase, NPW * C)], xr_l)
            pltpu.sync_copy(att_hbm.at[ci], att_v)

            def _zero_out(j, _):
                out_l[pl.ds(j * LANES, LANES)] = jnp.zeros(
                    (LANES,), jnp.float32)
                return 0
            lax.fori_loop(0, NPW * C // LANES, _zero_out, 0)

            def _init_md(j, _):
                m_l[pl.ds(j * LANES, LANES)] = jnp.full(
                    (LANES,), -3.0e38, jnp.float32)
                den_l[pl.ds(j * LANES, LANES)] = jnp.zeros(
                    (LANES,), jnp.float32)
                return 0
            lax.fori_loop(0, NPW, _init_md, 0)

            def _batch1(b, _):
                base = pl.multiple_of(a0 + b * EDGE_B, 8)
                pltpu.sync_copy(src_hbm.at[pl.ds(base, EDGE_B)], sidx)
                pltpu.sync_copy(dst_hbm.at[pl.ds(base, EDGE_B)],
                                didx.at[pl.ds(0, EDGE_B)])
                pltpu.async_copy(xl_hbm.at[ci].at[sidx], xl_b, sem).wait()
                lo = jnp.maximum(e0 - base, 0)
                hi = jnp.minimum(e1 - base, EDGE_B)

                def _edge1(i, _):
                    dloc = didx[pl.ds(i, LANES)][0] - n0
                    parts = []
                    for g in range(n_groups):
                        v = (xl_b[i, pl.ds(g * LANES, LANES)]
                             + xr_l[pl.ds(dloc * C + g * LANES, LANES)])
                        v = jnp.maximum(v, 0.0) + 0.2 * jnp.minimum(v, 0.0)
                        parts.append(v * att_v[pl.ds(g * LANES, LANES)])
                    lvec = jnp.zeros((LANES,), jnp.float32)
                    for hh in range(h_c):
                        acc = parts[hh * gph]
                        for g in range(1, gph):
                            acc = acc + parts[hh * gph + g]
                        lvec = jnp.where(iota == hh, jnp.sum(acc), lvec)
                    lg[i, :] = lvec
                    idx = dloc * LANES + iota
                    m_old = plsc.load_gather(m_l, [idx])
                    m_new = jnp.maximum(m_old, lvec)
                    scale = jnp.exp(m_old - m_new)
                    e = jnp.exp(lvec - m_new)
                    den_new = plsc.load_gather(den_l, [idx]) * scale + e
                    plsc.store_scatter(m_l, [idx], m_new, mask=lanemask)
                    plsc.store_scatter(den_l, [idx], den_new, mask=lanemask)
                    return 0
                lax.fori_loop(lo, hi, _edge1, 0)
                exrow = pl.multiple_of(exbase + b * EDGE_B, 8)
                pltpu.sync_copy(lg, ex_hbm.at[pl.ds(exrow, EDGE_B), :])
                return 0
            lax.fori_loop(0, nb, _batch1, 0)

            def _batch3(b, _):
                base = pl.multiple_of(a0 + b * EDGE_B, 8)
                pltpu.sync_copy(src_hbm.at[pl.ds(base, EDGE_B)], sidx)
                pltpu.sync_copy(dst_hbm.at[pl.ds(base, EDGE_B)],
                                didx.at[pl.ds(0, EDGE_B)])
                pltpu.async_copy(xl_hbm.at[ci].at[sidx], xl_b, sem).wait()
                exrow = pl.multiple_of(exbase + b * EDGE_B, 8)
                pltpu.sync_copy(ex_hbm.at[pl.ds(exrow, EDGE_B), :], lg)
                lo = jnp.maximum(e0 - base, 0)
                hi = jnp.minimum(e1 - base, EDGE_B)

                def _edge3(i, _):
                    dloc = didx[pl.ds(i, LANES)][0] - n0
                    idx = dloc * LANES + iota
                    den = plsc.load_gather(den_l, [idx])
                    mvec = plsc.load_gather(m_l, [idx])
                    ex = jnp.exp(lg[i, :] - mvec)
                    al[:] = ex / (den + 1e-16)
                    for g in range(n_groups):
                        head = (g * LANES) // dc
                        av = plsc.load_gather(
                            al, [jnp.full((LANES,), head, jnp.int32)])
                        contrib = xl_b[i, pl.ds(g * LANES, LANES)] * av
                        plsc.addupdate(
                            out_l.at[pl.ds(dloc * C + g * LANES, LANES)],
                            contrib)
                    return 0
                lax.fori_loop(lo, hi, _edge3, 0)
                return 0
            lax.fori_loop(0, nb, _batch3, 0)

            pltpu.sync_copy(out_l, out_hbm.at[ci, pl.ds(fbase, NPW * C)])

    return pl.kernel(
        body,
        out_type=(
            jax.ShapeDtypeStruct((nc, N_PAD * C), jnp.float32),
            jax.ShapeDtypeStruct((EX_ROWS, LANES), jnp.float32),
        ),
        mesh=mesh,
        compiler_params=pltpu.CompilerParams(
            needs_layout_passes=False, use_tc_tiling_on_sc=False),
        scratch_types=(
            pltpu.VMEM((NPW * C,), jnp.float32),
            pltpu.VMEM((NPW * C,), jnp.float32),
            pltpu.VMEM((EDGE_B, C), jnp.float32),
            pltpu.VMEM((NPW * LANES,), jnp.float32),
            pltpu.VMEM((NPW * LANES,), jnp.float32),
            pltpu.VMEM((EDGE_B, LANES), jnp.float32),
            pltpu.VMEM((LANES,), jnp.float32),
            pltpu.VMEM((128,), jnp.float32),
            pltpu.VMEM((EDGE_B,), jnp.int32),
            pltpu.VMEM((EDGE_B + LANES,), jnp.int32),
            pltpu.VMEM((56,), jnp.int32),
            pltpu.SemaphoreType.DMA,
        ),
    )


_CHUNKS = {
    (16, 32): (4, 4),
    (25, 16): (5, 5),
    (16, 16): (2, 8),
    (1, 32): (1, 1),
    (1, 128): (1, 1),
}


def _gat_layer(act, src_s, dst_s, ebounds, Wl, bl, Wr, br, att, bias):
    h, dc = att.shape
    hdc = h * dc
    nc, h_c = _CHUNKS[(h, dc)]
    C = h_c * dc

    W2 = jnp.concatenate([Wl, Wr], axis=1)
    b2 = jnp.concatenate([bl, br], axis=0)
    proj = _project(act, W2, b2, prev_bias=bias if bias is not None else None)

    proj = jnp.pad(proj, ((0, N_PAD - N_NODES), (0, 0)))
    xl = proj[:, :hdc].reshape(N_PAD, nc, C).transpose(1, 0, 2)
    xr = proj[:, hdc:].reshape(N_PAD, nc, C).transpose(1, 0, 2)
    att_e = att.reshape(nc, C)
    att_e = jnp.pad(att_e, ((0, 0), (0, 128 - C)))

    kern = _edge_kernel(h, dc, nc, h_c)
    out_ch, _ = kern(
        xl, xr.reshape(nc, N_PAD * C), att_e,
        src_s, dst_s, ebounds)
    out = out_ch.reshape(nc, N_PAD, C).transpose(1, 0, 2).reshape(N_PAD, hdc)
    return out[:N_NODES]


def kernel(x, adj, params):
    adj = adj.astype(jnp.int32)
    src, dst = adj[0], adj[1]

    combo = jnp.sort(dst * 16384 + src)
    dst_s = (combo >> 14).astype(jnp.int32)
    src_s = (combo & 16383).astype(jnp.int32)
    ebounds = jnp.searchsorted(
        dst_s, jnp.arange(NW + 1, dtype=jnp.int32) * NPW).astype(jnp.int32)
    ebounds = jnp.pad(ebounds, (0, 40 - (NW + 1)))
    src_s = jnp.pad(src_s, (0, E_PAD - N_EDGES))
    dst_s = jnp.pad(dst_s, (0, E_PAD - N_EDGES),
                    constant_values=N_NODES - 1)

    act = x
    prev_bias = None
    h_out = None
    for i in range(10):
        Wl, bl, Wr, br, att, bias = params[i]
        if i == 0:
            agg = _gat_layer(act, src_s, dst_s, ebounds,
                             Wl, bl, Wr, br, att, None)
        elif i == 5:
            agg = _gat_layer(h_out, src_s, dst_s, ebounds,
                             Wl, bl, Wr, br, att, None)
        else:
            agg = _gat_layer(agg, src_s, dst_s, ebounds,
                             Wl, bl, Wr, br, att, prev_bias)
        prev_bias = bias
        if i == 4:
            h_out = _softmax_bias(agg, bias)
        if i == 9:
            h1_out = _softmax_bias(agg, bias)
    return (h1_out, h_out)


# speedup vs baseline: 10.4704x; 1.4849x over previous
import functools

import jax
import jax.numpy as jnp
from jax import lax
from jax.experimental import pallas as pl
from jax.experimental.pallas import tpu as pltpu
from jax.experimental.pallas import tpu_sc as plsc

N_NODES = 10000
N_EDGES = 160000
NC, NS, LANES = 2, 16, 16
NW = NC * NS
NPW = 320
N_PAD = NW * NPW
EDGE_B = 64
E_PAD = N_EDGES + 2 * EDGE_B
EX_SPACING = EDGE_B + 16
MAX_RANGES = 128
EX_ROWS = N_EDGES + EX_SPACING * MAX_RANGES + 2 * EDGE_B


def _proj_body_plain(x_ref, w_ref, b_ref, o_ref):
    o_ref[...] = (
        jnp.dot(x_ref[...], w_ref[...], preferred_element_type=jnp.float32)
        + b_ref[...]
    )


def _proj_body_relu(x_ref, w_ref, b_ref, pb_ref, o_ref):
    a = jnp.maximum(x_ref[...] + pb_ref[...], 0.0)
    o_ref[...] = (
        jnp.dot(a, w_ref[...], preferred_element_type=jnp.float32) + b_ref[...]
    )


def _project(x, W, b, prev_bias=None, block_rows=1000):
    N, din = x.shape
    dout = W.shape[1]
    grid = (N // block_rows,)
    in_specs = [
        pl.BlockSpec((block_rows, din), lambda i: (i, 0)),
        pl.BlockSpec((din, dout), lambda i: (0, 0)),
        pl.BlockSpec((1, dout), lambda i: (0, 0)),
    ]
    args = [x, W, b.reshape(1, dout)]
    body = _proj_body_plain
    if prev_bias is not None:
        in_specs.append(pl.BlockSpec((1, din), lambda i: (0, 0)))
        args.append(prev_bias.reshape(1, din))
        body = _proj_body_relu
    return pl.pallas_call(
        body,
        grid=grid,
        in_specs=in_specs,
        out_specs=pl.BlockSpec((block_rows, dout), lambda i: (i, 0)),
        out_shape=jax.ShapeDtypeStruct((N, dout), jnp.float32),
    )(*args)


def _softmax_body(x_ref, b_ref, o_ref):
    z = x_ref[...] + b_ref[...]
    z = z - jnp.max(z, axis=1, keepdims=True)
    e = jnp.exp(z)
    o_ref[...] = e / jnp.sum(e, axis=1, keepdims=True)


def _softmax_bias(x, b, block_rows=1000):
    N, d = x.shape
    return pl.pallas_call(
        _softmax_body,
        grid=(N // block_rows,),
        in_specs=[
            pl.BlockSpec((block_rows, d), lambda i: (i, 0)),
            pl.BlockSpec((1, d), lambda i: (0, 0)),
        ],
        out_specs=pl.BlockSpec((block_rows, d), lambda i: (i, 0)),
        out_shape=jax.ShapeDtypeStruct((N, d), jnp.float32),
    )(x, b.reshape(1, d))


@functools.cache
def _edge_kernel(h, dc, ns, batch):
    C = h * dc
    NPS = NPW // ns
    n_groups = C // LANES
    gph = max(dc // LANES, 1)
    n_lvec = (h + LANES - 1) // LANES
    EW = n_lvec * LANES
    MD = 2 * LANES

    mesh = plsc.VectorSubcoreMesh(
        core_axis_name="c", subcore_axis_name="s",
        num_cores=NC, num_subcores=NS)

    def body(xl_hbm, xr_hbm, att_hbm, src_hbm, dst_hbm, ebnd_hbm,
             out_hbm, ex_hbm,
             xr_l, out_l, xl_b, m_l, den_l, lg, al, att_v, sidx, didx,
             ebv, sem):
        wid = lax.axis_index("s") * NC + lax.axis_index("c")
        iota = lax.broadcasted_iota(jnp.int32, (LANES,), 0)
        masks = [iota < (h - v * LANES) for v in range(n_lvec)]
        pltpu.sync_copy(ebnd_hbm, ebv.at[pl.ds(0, 136)])
        pltpu.sync_copy(att_hbm, att_v.at[pl.ds(0, C)])

        for s in range(ns):
            k = wid * ns + s
            n0 = k * NPS
            fbase = pl.multiple_of(n0 * C, 8)
            e0 = ebv[pl.ds(k, LANES)][0]
            e1 = ebv[pl.ds(k + 1, LANES)][0]
            a0 = (e0 // 8) * 8
            nb = (e1 - a0 + EDGE_B - 1) // EDGE_B
            exbase = ((e0 + 7) // 8) * 8 + EX_SPACING * k

            pltpu.sync_copy(xr_hbm.at[pl.ds(fbase, NPS * C)], xr_l)

            def _zero_out(j, _):
                out_l[pl.ds(j * LANES, LANES)] = jnp.zeros(
                    (LANES,), jnp.float32)
                return 0
            lax.fori_loop(0, NPS * C // LANES, _zero_out, 0)

            def _init_md(j, _):
                for v in range(2):
                    m_l[pl.ds(j * MD + v * LANES, LANES)] = jnp.full(
                        (LANES,), -3.0e38, jnp.float32)
                    den_l[pl.ds(j * MD + v * LANES, LANES)] = jnp.zeros(
                        (LANES,), jnp.float32)
                return 0
            lax.fori_loop(0, NPS, _init_md, 0)

            def _batch1(b, _):
                base = pl.multiple_of(a0 + b * EDGE_B, 8)
                pltpu.sync_copy(src_hbm.at[pl.ds(base, EDGE_B)], sidx)
                pltpu.sync_copy(dst_hbm.at[pl.ds(base, EDGE_B)],
                                didx.at[pl.ds(0, EDGE_B)])
                pltpu.async_copy(xl_hbm.at[sidx], xl_b, sem).wait()
                lo = jnp.maximum(e0 - base, 0)
                hi = jnp.minimum(e1 - base, EDGE_B)

                def _edge1(i, _):
                    dloc = didx[pl.ds(i, LANES)][0] - n0
                    parts = []
                    for g in range(n_groups):
                        v = (xl_b[i, pl.ds(g * LANES, LANES)]
                             + xr_l[pl.ds(dloc * C + g * LANES, LANES)])
                        v = jnp.maximum(v, 0.0) + 0.2 * jnp.minimum(v, 0.0)
                        parts.append(v * att_v[pl.ds(g * LANES, LANES)])
                    lvecs = []
                    for vv in range(n_lvec):
                        lvec = jnp.zeros((LANES,), jnp.float32)
                        for hl in range(min(LANES, h - vv * LANES)):
                            hh = vv * LANES + hl
                            acc = parts[hh * gph]
                            for g in range(1, gph):
                                acc = acc + parts[hh * gph + g]
                            lvec = jnp.where(iota == hl, jnp.sum(acc), lvec)
                        lvecs.append(lvec)
                        lg[i, pl.ds(vv * LANES, LANES)] = lvec
                    for vv in range(n_lvec):
                        idx = dloc * MD + vv * LANES + iota
                        m_old = plsc.load_gather(m_l, [idx])
                        m_new = jnp.maximum(m_old, lvecs[vv])
                        scale = jnp.exp(m_old - m_new)
                        e = jnp.exp(lvecs[vv] - m_new)
                        den_new = plsc.load_gather(den_l, [idx]) * scale + e
                        plsc.store_scatter(m_l, [idx], m_new, mask=masks[vv])
                        plsc.store_scatter(
                            den_l, [idx], den_new, mask=masks[vv])
                    return 0
                lax.fori_loop(lo, hi, _edge1, 0)
                exrow = pl.multiple_of(exbase + b * EDGE_B, 8)
                pltpu.sync_copy(lg, ex_hbm.at[pl.ds(exrow, EDGE_B), :])
                return 0
            lax.fori_loop(0, nb, _batch1, 0)

            def _batch3(b, _):
                base = pl.multiple_of(a0 + b * EDGE_B, 8)
                pltpu.sync_copy(src_hbm.at[pl.ds(base, EDGE_B)], sidx)
                pltpu.sync_copy(dst_hbm.at[pl.ds(base, EDGE_B)],
                                didx.at[pl.ds(0, EDGE_B)])
                pltpu.async_copy(xl_hbm.at[sidx], xl_b, sem).wait()
                exrow = pl.multiple_of(exbase + b * EDGE_B, 8)
                pltpu.sync_copy(ex_hbm.at[pl.ds(exrow, EDGE_B), :], lg)
                lo = jnp.maximum(e0 - base, 0)
                hi = jnp.minimum(e1 - base, EDGE_B)

                def _edge3(i, _):
                    dloc = didx[pl.ds(i, LANES)][0] - n0
                    for vv in range(n_lvec):
                        idx = dloc * MD + vv * LANES + iota
                        den = plsc.load_gather(den_l, [idx])
                        mvec = plsc.load_gather(m_l, [idx])
                        ex = jnp.exp(lg[i, pl.ds(vv * LANES, LANES)] - mvec)
                        al[pl.ds(vv * LANES, LANES)] = ex / (den + 1e-16)
                    for g in range(n_groups):
                        head = (g * LANES) // dc
                        av = plsc.load_gather(
                            al, [jnp.full((LANES,), head, jnp.int32)])
                        contrib = xl_b[i, pl.ds(g * LANES, LANES)] * av
                        plsc.addupdate(
                            out_l.at[pl.ds(dloc * C + g * LANES, LANES)],
                            contrib)
                    return 0
                lax.fori_loop(lo, hi, _edge3, 0)
                return 0
            lax.fori_loop(0, nb, _batch3, 0)

            pltpu.sync_copy(out_l, out_hbm.at[pl.ds(fbase, NPS * C)])

    return pl.kernel(
        body,
        out_type=(
            jax.ShapeDtypeStruct((N_PAD * C,), jnp.float32),
            jax.ShapeDtypeStruct((EX_ROWS, EW), jnp.float32),
        ),
        mesh=mesh,
        compiler_params=pltpu.CompilerParams(
            needs_layout_passes=False, use_tc_tiling_on_sc=False),
        scratch_types=(
            pltpu.VMEM((NPS * C,), jnp.float32),
            pltpu.VMEM((NPS * C,), jnp.float32),
            pltpu.VMEM((EDGE_B, C), jnp.float32),
            pltpu.VMEM((NPS * MD,), jnp.float32),
            pltpu.VMEM((NPS * MD,), jnp.float32),
            pltpu.VMEM((EDGE_B, EW), jnp.float32),
            pltpu.VMEM((2 * LANES,), jnp.float32),
            pltpu.VMEM((512,), jnp.float32),
            pltpu.VMEM((EDGE_B,), jnp.int32),
            pltpu.VMEM((EDGE_B + LANES,), jnp.int32),
            pltpu.VMEM((136 + LANES,), jnp.int32),
            pltpu.SemaphoreType.DMA,
        ),
    )


_NSLICE = {
    (16, 32): 4,
    (25, 16): 4,
    (16, 16): 2,
    (1, 32): 1,
    (1, 128): 1,
}


def _gat_layer(act, src_s, dst_s, ebounds, Wl, bl, Wr, br, att, prev_bias):
    h, dc = att.shape
    hdc = h * dc
    ns = _NSLICE[(h, dc)]

    W2 = jnp.concatenate([Wl, Wr], axis=1)
    b2 = jnp.concatenate([bl, br], axis=0)
    proj = _project(act, W2, b2, prev_bias=prev_bias)

    proj = jnp.pad(proj, ((0, N_PAD - N_NODES), (0, 0)))
    xl = proj[:, :hdc]
    xr = proj[:, hdc:]
    att_e = att.reshape(hdc)

    kern = _edge_kernel(h, dc, ns, EDGE_B)
    eb = ebounds[ns]
    out, _ = kern(xl, xr.reshape(N_PAD * hdc), att_e,
                  src_s, dst_s, eb)
    return out.reshape(N_PAD, hdc)[:N_NODES]


def kernel(x, adj, params):
    adj = adj.astype(jnp.int32)
    src, dst = adj[0], adj[1]

    combo = jnp.sort(dst * 16384 + src)
    dst_s = (combo >> 14).astype(jnp.int32)
    src_s = (combo & 16383).astype(jnp.int32)
    ebounds = {}
    for ns in (1, 2, 4):
        nps = NPW // ns
        eb = jnp.searchsorted(
            dst_s,
            jnp.arange(NW * ns + 1, dtype=jnp.int32) * nps).astype(jnp.int32)
        ebounds[ns] = jnp.pad(eb, (0, 136 - (NW * ns + 1)))
    src_s = jnp.pad(src_s, (0, E_PAD - N_EDGES))
    dst_s = jnp.pad(dst_s, (0, E_PAD - N_EDGES),
                    constant_values=N_NODES - 1)

    prev_bias = None
    h_out = None
    agg = None
    for i in range(10):
        Wl, bl, Wr, br, att, bias = params[i]
        if i == 0:
            agg = _gat_layer(x, src_s, dst_s, ebounds,
                             Wl, bl, Wr, br, att, None)
        elif i == 5:
            agg = _gat_layer(h_out, src_s, dst_s, ebounds,
                             Wl, bl, Wr, br, att, None)
        else:
            agg = _gat_layer(agg, src_s, dst_s, ebounds,
                             Wl, bl, Wr, br, att, prev_bias)
        prev_bias = bias
        if i == 4:
            h_out = _softmax_bias(agg, bias)
        if i == 9:
            h1_out = _softmax_bias(agg, bias)
    return (h1_out, h_out)
